# Initial kernel scaffold; baseline (speedup 1.0000x reference)
import jax
import jax.numpy as jnp
from jax.experimental import pallas as pl


def kernel(seq_ids, seq_len, table):
    raise NotImplementedError("write your pallas kernel here")


# speedup vs baseline: 7.8907x; 7.8907x over previous
import jax
import jax.numpy as jnp
from jax import lax
from jax.experimental import pallas as pl
from jax.experimental.pallas import tpu as pltpu
from jax.experimental.pallas import tpu_sc as plsc
import functools

_VOCAB = 100000
_D = 128
_B = 1024
_L = 200
_NC = 2
_NS = 16
_NW = _NC * _NS
_SEQ_PER_W = _B // _NW
_ROWS_PER_W = _SEQ_PER_W * _L
_G0 = 104
_G1 = _L - _G0


def _sc_body(ids_hbm, len_hbm, table_hbm, out_hbm,
             idx_v, len_v, buf0, buf1, g0, g1, o0, o1):
    bufs = (buf0, buf1)
    gsems = (g0, g1)
    osems = (o0, o1)
    wid = lax.axis_index("s") * _NC + lax.axis_index("c")
    base_row = wid * _ROWS_PER_W
    base_seq = wid * _SEQ_PER_W

    pltpu.sync_copy(ids_hbm.at[pl.ds(base_row, _ROWS_PER_W)], idx_v)
    pltpu.sync_copy(len_hbm.at[pl.ds(base_seq, _SEQ_PER_W)],
                    len_v.at[pl.ds(0, _SEQ_PER_W)])

    def fire_gather(s, b):
        off = s * _L
        pltpu.async_copy(table_hbm.at[idx_v.at[pl.ds(off, _G0)]],
                         bufs[b].at[pl.ds(0, _G0)], gsems[b])
        pltpu.async_copy(table_hbm.at[idx_v.at[pl.ds(off + _G0, _G1)]],
                         bufs[b].at[pl.ds(_G0, _G1)], gsems[b])

    def wait_gather(b):
        pltpu.make_async_copy(table_hbm.at[idx_v.at[pl.ds(0, _G0)]],
                              bufs[b].at[pl.ds(0, _G0)], gsems[b]).wait()
        pltpu.make_async_copy(table_hbm.at[idx_v.at[pl.ds(0, _G1)]],
                              bufs[b].at[pl.ds(_G0, _G1)], gsems[b]).wait()

    def fire_out(s, b):
        pltpu.async_copy(bufs[b], out_hbm.at[pl.ds(base_row + s * _L, _L)],
                         osems[b])

    def wait_out(b):
        pltpu.make_async_copy(bufs[b], out_hbm.at[pl.ds(base_row, _L)],
                              osems[b]).wait()

    zeros16 = jnp.zeros((16,), jnp.float32)

    def zero_tail(s, b):
        ln = len_v[pl.ds(s, 16)][0]

        @pl.loop(ln, _L)
        def _(r):
            for c in range(_D // 16):
                bufs[b][r, pl.ds(c * 16, 16)] = zeros16

    fire_gather(0, 0)
    fire_gather(1, 1)
    wait_gather(0)
    zero_tail(0, 0)
    fire_out(0, 0)

    @pl.loop(1, _SEQ_PER_W - 1, step=2)
    def _(s0):
        for j in range(2):
            s = s0 + j
            p = (1 + j) % 2
            q = 1 - p
            wait_out(q)
            fire_gather(s + 1, q)
            wait_gather(p)
            zero_tail(s, p)
            fire_out(s, p)

    wait_out(0)
    wait_gather(1)
    zero_tail(_SEQ_PER_W - 1, 1)
    fire_out(_SEQ_PER_W - 1, 1)
    wait_out(1)


@jax.jit
def kernel(seq_ids, seq_len, table):
    ids_flat = seq_ids.reshape(-1).astype(jnp.int32)
    len32 = seq_len.astype(jnp.int32)
    table = table.astype(jnp.float32)

    mesh = plsc.VectorSubcoreMesh(core_axis_name="c", subcore_axis_name="s")
    run = pl.kernel(
        _sc_body,
        out_type=jax.ShapeDtypeStruct((_B * _L, _D), jnp.float32),
        mesh=mesh,
        scratch_types=[
            pltpu.VMEM((_ROWS_PER_W,), jnp.int32),
            pltpu.VMEM((_SEQ_PER_W + 16, ), jnp.int32),
            pltpu.VMEM((_L, _D), jnp.float32),
            pltpu.VMEM((_L, _D), jnp.float32),
            pltpu.SemaphoreType.DMA,
            pltpu.SemaphoreType.DMA,
            pltpu.SemaphoreType.DMA,
            pltpu.SemaphoreType.DMA,
        ],
    )
    out = run(ids_flat, len32, table)
    return out.reshape(_B, _L, _D)


# speedup vs baseline: 8.9372x; 1.1326x over previous
import jax
import jax.numpy as jnp
from jax import lax
from jax.experimental import pallas as pl
from jax.experimental.pallas import tpu as pltpu
from jax.experimental.pallas import tpu_sc as plsc
import functools

_VOCAB = 100000
_D = 128
_B = 1024
_L = 200
_NC = 2
_NS = 16
_NW = _NC * _NS
_SEQ_PER_W = _B // _NW
_ROWS_PER_W = _SEQ_PER_W * _L
_GC = 40
_NCHUNK = _L // _GC


def _sc_body(ids_hbm, len_hbm, table_hbm, out_hbm,
             idx_v, len_v, buf0, buf1, g0, g1, o0, o1):
    bufs = (buf0, buf1)
    gsems = (g0, g1)
    osems = (o0, o1)
    wid = lax.axis_index("s") * _NC + lax.axis_index("c")
    base_row = wid * _ROWS_PER_W
    base_seq = wid * _SEQ_PER_W

    pltpu.sync_copy(ids_hbm.at[pl.ds(base_row, _ROWS_PER_W)], idx_v)
    pltpu.sync_copy(len_hbm.at[pl.ds(base_seq, _SEQ_PER_W)],
                    len_v.at[pl.ds(0, _SEQ_PER_W)])

    def seq_len_at(s):
        return len_v[pl.ds(s, 16)][0]

    def fire_gather(s, b):
        off = s * _L
        ln = seq_len_at(s)
        for c in range(_NCHUNK):
            @pl.when(ln > c * _GC)
            def _():
                pltpu.async_copy(
                    table_hbm.at[idx_v.at[pl.ds(off + c * _GC, _GC)]],
                    bufs[b].at[pl.ds(c * _GC, _GC)], gsems[b])

    def wait_gather(s, b):
        ln = seq_len_at(s)
        for c in range(_NCHUNK):
            @pl.when(ln > c * _GC)
            def _():
                pltpu.make_async_copy(
                    table_hbm.at[idx_v.at[pl.ds(0, _GC)]],
                    bufs[b].at[pl.ds(c * _GC, _GC)], gsems[b]).wait()

    def fire_out(s, b):
        pltpu.async_copy(bufs[b], out_hbm.at[pl.ds(base_row + s * _L, _L)],
                         osems[b])

    def wait_out(b):
        pltpu.make_async_copy(bufs[b], out_hbm.at[pl.ds(base_row, _L)],
                              osems[b]).wait()

    zeros16 = jnp.zeros((16,), jnp.float32)

    def zero_tail(s, b):
        ln = seq_len_at(s)

        @pl.loop(ln, _L)
        def _(r):
            for c in range(_D // 16):
                bufs[b][r, pl.ds(c * 16, 16)] = zeros16

    fire_gather(0, 0)
    fire_gather(1, 1)
    wait_gather(0, 0)
    zero_tail(0, 0)
    fire_out(0, 0)

    @pl.loop(1, _SEQ_PER_W - 1, step=2)
    def _(s0):
        for j in range(2):
            s = s0 + j
            p = (1 + j) % 2
            q = 1 - p
            wait_out(q)
            fire_gather(s + 1, q)
            wait_gather(s, p)
            zero_tail(s, p)
            fire_out(s, p)

    wait_out(0)
    wait_gather(_SEQ_PER_W - 1, 1)
    zero_tail(_SEQ_PER_W - 1, 1)
    fire_out(_SEQ_PER_W - 1, 1)
    wait_out(1)


@jax.jit
def kernel(seq_ids, seq_len, table):
    ids_flat = seq_ids.reshape(-1).astype(jnp.int32)
    len32 = seq_len.astype(jnp.int32)
    table = table.astype(jnp.float32)

    mesh = plsc.VectorSubcoreMesh(core_axis_name="c", subcore_axis_name="s")
    run = pl.kernel(
        _sc_body,
        out_type=jax.ShapeDtypeStruct((_B * _L, _D), jnp.float32),
        mesh=mesh,
        scratch_types=[
            pltpu.VMEM((_ROWS_PER_W,), jnp.int32),
            pltpu.VMEM((_SEQ_PER_W + 16, ), jnp.int32),
            pltpu.VMEM((_L, _D), jnp.float32),
            pltpu.VMEM((_L, _D), jnp.float32),
            pltpu.SemaphoreType.DMA,
            pltpu.SemaphoreType.DMA,
            pltpu.SemaphoreType.DMA,
            pltpu.SemaphoreType.DMA,
        ],
    )
    out = run(ids_flat, len32, table)
    return out.reshape(_B, _L, _D)


# speedup vs baseline: 9.1935x; 1.0287x over previous
import jax
import jax.numpy as jnp
from jax import lax
from jax.experimental import pallas as pl
from jax.experimental.pallas import tpu as pltpu
from jax.experimental.pallas import tpu_sc as plsc
import functools

_VOCAB = 100000
_D = 128
_B = 1024
_L = 200
_NC = 2
_NS = 16
_NW = _NC * _NS
_SEQ_PER_W = _B // _NW
_ROWS_PER_W = _SEQ_PER_W * _L
_GSIZES = (128, 64, 32, 16, 8)


def _sc_body(ids_hbm, len_hbm, table_hbm, out_hbm,
             idx_v, len_v, buf0, buf1, g0, g1, o0, o1):
    bufs = (buf0, buf1)
    gsems = (g0, g1)
    osems = (o0, o1)
    wid = lax.axis_index("s") * _NC + lax.axis_index("c")
    base_row = wid * _ROWS_PER_W
    base_seq = wid * _SEQ_PER_W

    pltpu.sync_copy(ids_hbm.at[pl.ds(base_row, _ROWS_PER_W)], idx_v)
    pltpu.sync_copy(len_hbm.at[pl.ds(base_seq, _SEQ_PER_W)],
                    len_v.at[pl.ds(0, _SEQ_PER_W)])

    def seq_len_at(s):
        return len_v[pl.ds(s, 16)][0]

    def fire_gather(s, b):
        off = s * _L
        lnu = (seq_len_at(s) + 7) & ~7
        acc = 0
        for sz in _GSIZES:
            part = lnu & sz
            at = acc

            @pl.when(part != 0)
            def _():
                pltpu.async_copy(
                    table_hbm.at[idx_v.at[pl.ds(pl.multiple_of(off + at, 8), sz)]],
                    bufs[b].at[pl.ds(pl.multiple_of(at, 8), sz)], gsems[b])
            acc = acc + part

    def wait_gather(s, b):
        lnu = (seq_len_at(s) + 7) & ~7
        for sz in _GSIZES:
            @pl.when((lnu & sz) != 0)
            def _():
                pltpu.make_async_copy(
                    table_hbm.at[idx_v.at[pl.ds(0, sz)]],
                    bufs[b].at[pl.ds(0, sz)], gsems[b]).wait()

    def fire_out(s, b):
        pltpu.async_copy(bufs[b], out_hbm.at[pl.ds(base_row + s * _L, _L)],
                         osems[b])

    def wait_out(b):
        pltpu.make_async_copy(bufs[b], out_hbm.at[pl.ds(base_row, _L)],
                              osems[b]).wait()

    zeros16 = jnp.zeros((16,), jnp.float32)

    def zero_tail(s, b):
        ln = seq_len_at(s)

        @pl.loop(ln, _L)
        def _(r):
            for c in range(_D // 16):
                bufs[b][r, pl.ds(c * 16, 16)] = zeros16

    fire_gather(0, 0)
    fire_gather(1, 1)
    wait_gather(0, 0)
    zero_tail(0, 0)
    fire_out(0, 0)

    @pl.loop(1, _SEQ_PER_W - 1, step=2)
    def _(s0):
        for j in range(2):
            s = s0 + j
            p = (1 + j) % 2
            q = 1 - p
            wait_out(q)
            fire_gather(s + 1, q)
            wait_gather(s, p)
            zero_tail(s, p)
            fire_out(s, p)

    wait_out(0)
    wait_gather(_SEQ_PER_W - 1, 1)
    zero_tail(_SEQ_PER_W - 1, 1)
    fire_out(_SEQ_PER_W - 1, 1)
    wait_out(1)


@jax.jit
def kernel(seq_ids, seq_len, table):
    ids_flat = seq_ids.reshape(-1).astype(jnp.int32)
    len32 = seq_len.astype(jnp.int32)
    table = table.astype(jnp.float32)

    mesh = plsc.VectorSubcoreMesh(core_axis_name="c", subcore_axis_name="s")
    run = pl.kernel(
        _sc_body,
        out_type=jax.ShapeDtypeStruct((_B * _L, _D), jnp.float32),
        mesh=mesh,
        scratch_types=[
            pltpu.VMEM((_ROWS_PER_W,), jnp.int32),
            pltpu.VMEM((_SEQ_PER_W + 16, ), jnp.int32),
            pltpu.VMEM((_L, _D), jnp.float32),
            pltpu.VMEM((_L, _D), jnp.float32),
            pltpu.SemaphoreType.DMA,
            pltpu.SemaphoreType.DMA,
            pltpu.SemaphoreType.DMA,
            pltpu.SemaphoreType.DMA,
        ],
    )
    out = run(ids_flat, len32, table)
    return out.reshape(_B, _L, _D)


# speedup vs baseline: 9.3235x; 1.0141x over previous
import jax
import jax.numpy as jnp
from jax import lax
from jax.experimental import pallas as pl
from jax.experimental.pallas import tpu as pltpu
from jax.experimental.pallas import tpu_sc as plsc
import functools

_VOCAB = 100000
_D = 128
_B = 1024
_L = 200
_NC = 2
_NS = 16
_NW = _NC * _NS
_SEQ_PER_W = _B // _NW
_ROWS_PER_W = _SEQ_PER_W * _L
_GSIZES = (128, 64, 32, 16, 8)


def _sc_body(ids_hbm, len_hbm, table_hbm, out_hbm,
             idx_v, len_v, buf0, buf1, buf2, buf3,
             g0, g1, g2, g3, o0, o1, o2, o3):
    bufs = (buf0, buf1, buf2, buf3)
    gsems = (g0, g1, g2, g3)
    osems = (o0, o1, o2, o3)
    wid = lax.axis_index("s") * _NC + lax.axis_index("c")
    base_row = wid * _ROWS_PER_W
    base_seq = wid * _SEQ_PER_W

    pltpu.sync_copy(ids_hbm.at[pl.ds(base_row, _ROWS_PER_W)], idx_v)
    pltpu.sync_copy(len_hbm.at[pl.ds(base_seq, _SEQ_PER_W)],
                    len_v.at[pl.ds(0, _SEQ_PER_W)])

    def seq_len_at(s):
        return len_v[pl.ds(s, 16)][0]

    def fire_gather(s, b):
        off = s * _L
        lnu = (seq_len_at(s) + 7) & ~7
        acc = 0
        for sz in _GSIZES:
            part = lnu & sz
            at = acc

            @pl.when(part != 0)
            def _():
                pltpu.async_copy(
                    table_hbm.at[idx_v.at[pl.ds(pl.multiple_of(off + at, 8), sz)]],
                    bufs[b].at[pl.ds(pl.multiple_of(at, 8), sz)], gsems[b])
            acc = acc + part

    def wait_gather(s, b):
        lnu = (seq_len_at(s) + 7) & ~7
        for sz in _GSIZES:
            @pl.when((lnu & sz) != 0)
            def _():
                pltpu.make_async_copy(
                    table_hbm.at[idx_v.at[pl.ds(0, sz)]],
                    bufs[b].at[pl.ds(0, sz)], gsems[b]).wait()

    def fire_out(s, b):
        pltpu.async_copy(bufs[b], out_hbm.at[pl.ds(base_row + s * _L, _L)],
                         osems[b])

    def wait_out(b):
        pltpu.make_async_copy(bufs[b], out_hbm.at[pl.ds(base_row, _L)],
                              osems[b]).wait()

    zeros16 = jnp.zeros((16,), jnp.float32)

    def zero_tail(s, b):
        ln = seq_len_at(s)

        @pl.loop(ln, _L)
        def _(r):
            for c in range(_D // 16):
                bufs[b][r, pl.ds(c * 16, 16)] = zeros16

    fire_gather(0, 0)
    fire_gather(1, 1)
    for s in (0, 1):
        wait_gather(s, s)
        zero_tail(s, s)
        fire_out(s, s)
        fire_gather(s + 2, s + 2)

    @pl.loop(2, _SEQ_PER_W - 2, step=4)
    def _(s0):
        for j in range(4):
            s = s0 + j
            p = (2 + j) % 4
            q = j
            wait_gather(s, p)
            zero_tail(s, p)
            fire_out(s, p)
            wait_out(q)
            fire_gather(s + 2, q)

    for s, p in ((_SEQ_PER_W - 2, 2), (_SEQ_PER_W - 1, 3)):
        wait_gather(s, p)
        zero_tail(s, p)
        fire_out(s, p)
        wait_out(p - 2)
    wait_out(2)
    wait_out(3)


@jax.jit
def kernel(seq_ids, seq_len, table):
    ids_flat = seq_ids.reshape(-1).astype(jnp.int32)
    len32 = seq_len.astype(jnp.int32)
    table = table.astype(jnp.float32)

    mesh = plsc.VectorSubcoreMesh(core_axis_name="c", subcore_axis_name="s")
    run = pl.kernel(
        _sc_body,
        out_type=jax.ShapeDtypeStruct((_B * _L, _D), jnp.float32),
        mesh=mesh,
        scratch_types=[
            pltpu.VMEM((_ROWS_PER_W,), jnp.int32),
            pltpu.VMEM((_SEQ_PER_W + 16, ), jnp.int32),
            pltpu.VMEM((_L, _D), jnp.float32),
            pltpu.VMEM((_L, _D), jnp.float32),
            pltpu.VMEM((_L, _D), jnp.float32),
            pltpu.VMEM((_L, _D), jnp.float32),
            pltpu.SemaphoreType.DMA,
            pltpu.SemaphoreType.DMA,
            pltpu.SemaphoreType.DMA,
            pltpu.SemaphoreType.DMA,
            pltpu.SemaphoreType.DMA,
            pltpu.SemaphoreType.DMA,
            pltpu.SemaphoreType.DMA,
            pltpu.SemaphoreType.DMA,
        ],
    )
    out = run(ids_flat, len32, table)
    return out.reshape(_B, _L, _D)
